# Initial kernel scaffold; baseline (speedup 1.0000x reference)
import jax
import jax.numpy as jnp
from jax.experimental import pallas as pl


def kernel(x_generator, x_bus, x_reserve, ei_produces_at, ea_produces_at, ei_served_by, ea_served_by, ei_transmission, ea_transmission, ei_backed_by, ea_backed_by, proj_w_generator, proj_b_generator, proj_w_bus, proj_b_bus, proj_w_reserve, proj_b_reserve, hl_w_1, hl_b_1, ete_1, eae_w_1, att_w_1, lin_w_1, lin_b_1, hl_w_2, hl_b_2, ete_2, eae_w_2, att_w_2, lin_w_2, lin_b_2, hl_w_3, hl_b_3, ete_3, eae_w_3, att_w_3, lin_w_3, lin_b_3):
    raise NotImplementedError("write your pallas kernel here")


# speedup vs baseline: 1.5917x; 1.5917x over previous
import jax
import jax.numpy as jnp
from jax.experimental import pallas as pl

_N_GEN, _N_BUS, _N_RES = 20000, 70000, 10000
_N = _N_GEN + _N_BUS + _N_RES
_E_LIST = (80000, 80000, 160000, 80000)
_E = sum(_E_LIST)
_HID = 128
_HEADS = 4
_EEMB = 16
_NEG = 0.2

_NODE_TILE = 2000
_EDGE_TILE = 4000


def _leaky(x):
    return jnp.where(x >= 0, x, _NEG * x)


def _mm_bias(x, w, b, tile):
    m, k = x.shape
    h = w.shape[1]

    def kern(x_ref, w_ref, b_ref, o_ref):
        o_ref[...] = (
            jnp.dot(x_ref[...], w_ref[...], preferred_element_type=jnp.float32)
            + b_ref[...]
        )

    return pl.pallas_call(
        kern,
        grid=(m // tile,),
        in_specs=[
            pl.BlockSpec((tile, k), lambda i: (i, 0)),
            pl.BlockSpec((k, h), lambda i: (0, 0)),
            pl.BlockSpec((1, h), lambda i: (0, 0)),
        ],
        out_specs=pl.BlockSpec((tile, h), lambda i: (i, 0)),
        out_shape=jax.ShapeDtypeStruct((m, h), jnp.float32),
    )(x, w, b.reshape(1, h))


def _hetero_linear(x, hl_w, hl_b):
    parts = []
    for t, (lo, hi) in enumerate(
        ((0, _N_GEN), (_N_GEN, _N_GEN + _N_BUS), (_N_GEN + _N_BUS, _N))
    ):
        parts.append(_mm_bias(x[lo:hi], hl_w[t], hl_b[t], _NODE_TILE))
    return jnp.concatenate(parts, axis=0)


def _edge_stage(x_i, x_j, et, edge_attr, eae_w, att_w, lin_w, lin_b):
    aw1 = att_w[0:_HID]
    aw2 = att_w[_HID : 2 * _HID]
    aw3 = att_w[2 * _HID : 2 * _HID + _EEMB]
    aw4 = att_w[2 * _HID + _EEMB :]
    lw1 = lin_w[0:_HID]
    lw2 = lin_w[_HID:]
    h8 = 8
    pad = h8 - _HEADS
    aw1 = jnp.pad(aw1, ((0, 0), (0, pad)))
    aw2 = jnp.pad(aw2, ((0, 0), (0, pad)))
    aw3 = jnp.pad(aw3, ((0, 0), (0, pad)))
    aw4 = jnp.pad(aw4, ((0, 0), (0, pad)))

    def kern(xi_ref, xj_ref, et_ref, ea_ref, aw1_ref, aw2_ref, aw3_ref,
             aw4_ref, ew_ref, lw1_ref, lw2_ref, lb_ref, alpha_ref, msg_ref):
        ea = _leaky(jnp.dot(ea_ref[...], ew_ref[...],
                            preferred_element_type=jnp.float32))
        xi = xi_ref[...]
        xj = xj_ref[...]
        a = (
            jnp.dot(xi, aw1_ref[...], preferred_element_type=jnp.float32)
            + jnp.dot(xj, aw2_ref[...], preferred_element_type=jnp.float32)
            + jnp.dot(et_ref[...], aw3_ref[...], preferred_element_type=jnp.float32)
            + jnp.dot(ea, aw4_ref[...], preferred_element_type=jnp.float32)
        )
        alpha_ref[...] = _leaky(a)
        msg_ref[...] = (
            jnp.dot(xj, lw1_ref[...], preferred_element_type=jnp.float32)
            + jnp.dot(ea, lw2_ref[...], preferred_element_type=jnp.float32)
            + lb_ref[...]
        )

    t = _EDGE_TILE
    edim = edge_attr.shape[1]
    alpha, msg = pl.pallas_call(
        kern,
        grid=(_E // t,),
        in_specs=[
            pl.BlockSpec((t, _HID), lambda i: (i, 0)),
            pl.BlockSpec((t, _HID), lambda i: (i, 0)),
            pl.BlockSpec((t, _EEMB), lambda i: (i, 0)),
            pl.BlockSpec((t, edim), lambda i: (i, 0)),
            pl.BlockSpec((_HID, h8), lambda i: (0, 0)),
            pl.BlockSpec((_HID, h8), lambda i: (0, 0)),
            pl.BlockSpec((_EEMB, h8), lambda i: (0, 0)),
            pl.BlockSpec((_EEMB, h8), lambda i: (0, 0)),
            pl.BlockSpec((edim, _EEMB), lambda i: (0, 0)),
            pl.BlockSpec((_HID, _HID), lambda i: (0, 0)),
            pl.BlockSpec((_EEMB, _HID), lambda i: (0, 0)),
            pl.BlockSpec((1, _HID), lambda i: (0, 0)),
        ],
        out_specs=[
            pl.BlockSpec((t, h8), lambda i: (i, 0)),
            pl.BlockSpec((t, _HID), lambda i: (i, 0)),
        ],
        out_shape=[
            jax.ShapeDtypeStruct((_E, h8), jnp.float32),
            jax.ShapeDtypeStruct((_E, _HID), jnp.float32),
        ],
    )(x_i, x_j, et, edge_attr, aw1, aw2, aw3, aw4, eae_w, lw1, lw2,
      lin_b.reshape(1, _HID))
    return alpha[:, :_HEADS], msg


def _heat_conv(x, src, dst, et, edge_attr, hl_w, hl_b, eae_w, att_w,
               lin_w, lin_b):
    xw = _hetero_linear(x, hl_w, hl_b)
    x_j = jnp.take(xw, src, axis=0)
    x_i = jnp.take(xw, dst, axis=0)
    alpha, msg = _edge_stage(x_i, x_j, et, edge_attr, eae_w, att_w,
                             lin_w, lin_b)
    amax = jax.ops.segment_max(alpha, dst, num_segments=_N)
    amax = jnp.where(jnp.isfinite(amax), amax, 0.0)
    ae = jnp.exp(alpha - jnp.take(amax, dst, axis=0))
    asum = jax.ops.segment_sum(ae, dst, num_segments=_N)
    attn = ae / (jnp.take(asum, dst, axis=0) + 1e-16)
    w_edge = attn.sum(axis=1, keepdims=True)
    acc = jax.ops.segment_sum(msg * w_edge, dst, num_segments=_N)
    return acc / float(_HEADS)


def kernel(x_generator, x_bus, x_reserve, ei_produces_at, ea_produces_at,
           ei_served_by, ea_served_by, ei_transmission, ea_transmission,
           ei_backed_by, ea_backed_by, proj_w_generator, proj_b_generator,
           proj_w_bus, proj_b_bus, proj_w_reserve, proj_b_reserve,
           hl_w_1, hl_b_1, ete_1, eae_w_1, att_w_1, lin_w_1, lin_b_1,
           hl_w_2, hl_b_2, ete_2, eae_w_2, att_w_2, lin_w_2, lin_b_2,
           hl_w_3, hl_b_3, ete_3, eae_w_3, att_w_3, lin_w_3, lin_b_3):
    xg = _mm_bias(x_generator, proj_w_generator, proj_b_generator, _NODE_TILE)
    xb = _mm_bias(x_bus, proj_w_bus, proj_b_bus, _NODE_TILE)
    xr = _mm_bias(x_reserve, proj_w_reserve, proj_b_reserve, _NODE_TILE)
    x = jnp.concatenate([xg, xb, xr], axis=0)

    edge_index = jnp.concatenate(
        [ei_produces_at, ei_served_by, ei_transmission, ei_backed_by], axis=1)
    src = edge_index[0]
    dst = edge_index[1]
    edge_attr = jnp.concatenate(
        [ea_produces_at, ea_served_by, ea_transmission, ea_backed_by], axis=0)

    def et_full(ete):
        tbl = _leaky(ete)
        return jnp.concatenate(
            [jnp.broadcast_to(tbl[i], (e, _EEMB)) for i, e in enumerate(_E_LIST)],
            axis=0)

    h1 = jax.nn.relu(_heat_conv(x, src, dst, et_full(ete_1), edge_attr,
                                hl_w_1, hl_b_1, eae_w_1, att_w_1,
                                lin_w_1, lin_b_1))
    h2 = jax.nn.relu(_heat_conv(h1, src, dst, et_full(ete_2), edge_attr,
                                hl_w_2, hl_b_2, eae_w_2, att_w_2,
                                lin_w_2, lin_b_2)) + h1
    h3 = jax.nn.relu(_heat_conv(h2, src, dst, et_full(ete_3), edge_attr,
                                hl_w_3, hl_b_3, eae_w_3, att_w_3,
                                lin_w_3, lin_b_3)) + h2

    node_type = jnp.concatenate([
        jnp.zeros((_N_GEN,), jnp.int32),
        jnp.ones((_N_BUS,), jnp.int32),
        jnp.full((_N_RES,), 2, jnp.int32),
    ])
    return (h3, node_type)


# speedup vs baseline: 1.6165x; 1.0156x over previous
import jax
import jax.numpy as jnp
from jax.experimental import pallas as pl

_N_GEN, _N_BUS, _N_RES = 20000, 70000, 10000
_N = _N_GEN + _N_BUS + _N_RES
_E_LIST = (80000, 80000, 160000, 80000)
_E = sum(_E_LIST)
_HID = 128
_HEADS = 4
_EEMB = 16
_NEG = 0.2

_NODE_TILE = 2000
_EDGE_TILE = 4000


def _leaky(x):
    return jnp.where(x >= 0, x, _NEG * x)


def _mm_bias(x, w, b, tile):
    m, k = x.shape
    h = w.shape[1]

    def kern(x_ref, w_ref, b_ref, o_ref):
        o_ref[...] = (
            jnp.dot(x_ref[...], w_ref[...], preferred_element_type=jnp.float32)
            + b_ref[...]
        )

    return pl.pallas_call(
        kern,
        grid=(m // tile,),
        in_specs=[
            pl.BlockSpec((tile, k), lambda i: (i, 0)),
            pl.BlockSpec((k, h), lambda i: (0, 0)),
            pl.BlockSpec((1, h), lambda i: (0, 0)),
        ],
        out_specs=pl.BlockSpec((tile, h), lambda i: (i, 0)),
        out_shape=jax.ShapeDtypeStruct((m, h), jnp.float32),
    )(x, w, b.reshape(1, h))


def _hetero_linear(x, hl_w, hl_b):
    parts = []
    for t, (lo, hi) in enumerate(
        ((0, _N_GEN), (_N_GEN, _N_GEN + _N_BUS), (_N_GEN + _N_BUS, _N))
    ):
        parts.append(_mm_bias(x[lo:hi], hl_w[t], hl_b[t], _NODE_TILE))
    return jnp.concatenate(parts, axis=0)


def _edge_stage(x_i, x_j, ete, edge_attr, eae_w, att_w, lin_w, lin_b):
    aw1 = att_w[0:_HID]
    aw2 = att_w[_HID : 2 * _HID]
    aw3 = att_w[2 * _HID : 2 * _HID + _EEMB]
    aw4 = att_w[2 * _HID + _EEMB :]
    lw1 = lin_w[0:_HID]
    lw2 = lin_w[_HID:]
    h8 = 8
    pad = h8 - _HEADS
    aw1 = jnp.pad(aw1, ((0, 0), (0, pad)))
    aw2 = jnp.pad(aw2, ((0, 0), (0, pad)))
    aw4 = jnp.pad(aw4, ((0, 0), (0, pad)))
    etl = jnp.pad(_leaky(ete) @ aw3, ((0, 0), (0, pad)))
    tile_type = jnp.repeat(
        jnp.arange(4, dtype=jnp.int32),
        jnp.array([e // _EDGE_TILE for e in _E_LIST]),
        total_repeat_length=_E // _EDGE_TILE)
    tile_logit = jnp.take(etl, tile_type, axis=0)[:, None, :]

    def kern(xi_ref, xj_ref, etl_ref, ea_ref, aw1_ref, aw2_ref,
             aw4_ref, ew_ref, lw1_ref, lw2_ref, lb_ref, alpha_ref, msg_ref):
        ea = _leaky(jnp.dot(ea_ref[...], ew_ref[...],
                            preferred_element_type=jnp.float32))
        xi = xi_ref[...]
        xj = xj_ref[...]
        a = (
            jnp.dot(xi, aw1_ref[...], preferred_element_type=jnp.float32)
            + jnp.dot(xj, aw2_ref[...], preferred_element_type=jnp.float32)
            + etl_ref[0]
            + jnp.dot(ea, aw4_ref[...], preferred_element_type=jnp.float32)
        )
        alpha_ref[...] = _leaky(a)
        msg_ref[...] = (
            jnp.dot(xj, lw1_ref[...], preferred_element_type=jnp.float32)
            + jnp.dot(ea, lw2_ref[...], preferred_element_type=jnp.float32)
            + lb_ref[...]
        )

    t = _EDGE_TILE
    edim = edge_attr.shape[1]
    alpha, msg = pl.pallas_call(
        kern,
        grid=(_E // t,),
        in_specs=[
            pl.BlockSpec((t, _HID), lambda i: (i, 0)),
            pl.BlockSpec((t, _HID), lambda i: (i, 0)),
            pl.BlockSpec((1, 1, h8), lambda i: (i, 0, 0)),
            pl.BlockSpec((t, edim), lambda i: (i, 0)),
            pl.BlockSpec((_HID, h8), lambda i: (0, 0)),
            pl.BlockSpec((_HID, h8), lambda i: (0, 0)),
            pl.BlockSpec((_EEMB, h8), lambda i: (0, 0)),
            pl.BlockSpec((edim, _EEMB), lambda i: (0, 0)),
            pl.BlockSpec((_HID, _HID), lambda i: (0, 0)),
            pl.BlockSpec((_EEMB, _HID), lambda i: (0, 0)),
            pl.BlockSpec((1, _HID), lambda i: (0, 0)),
        ],
        out_specs=[
            pl.BlockSpec((t, h8), lambda i: (i, 0)),
            pl.BlockSpec((t, _HID), lambda i: (i, 0)),
        ],
        out_shape=[
            jax.ShapeDtypeStruct((_E, h8), jnp.float32),
            jax.ShapeDtypeStruct((_E, _HID), jnp.float32),
        ],
    )(x_i, x_j, tile_logit, edge_attr, aw1, aw2, aw4, eae_w, lw1, lw2,
      lin_b.reshape(1, _HID))
    return alpha[:, :_HEADS], msg


def _heat_conv(x, src, dst, ete, edge_attr, hl_w, hl_b, eae_w, att_w,
               lin_w, lin_b):
    xw = _hetero_linear(x, hl_w, hl_b)
    x_j = jnp.take(xw, src, axis=0)
    x_i = jnp.take(xw, dst, axis=0)
    alpha, msg = _edge_stage(x_i, x_j, ete, edge_attr, eae_w, att_w,
                             lin_w, lin_b)
    amax = jax.ops.segment_max(alpha, dst, num_segments=_N)
    amax = jnp.where(jnp.isfinite(amax), amax, 0.0)
    ae = jnp.exp(alpha - jnp.take(amax, dst, axis=0))
    asum = jax.ops.segment_sum(ae, dst, num_segments=_N)
    attn = ae / (jnp.take(asum, dst, axis=0) + 1e-16)
    w_edge = attn.sum(axis=1, keepdims=True)
    acc = jax.ops.segment_sum(msg * w_edge, dst, num_segments=_N)
    return acc / float(_HEADS)


def kernel(x_generator, x_bus, x_reserve, ei_produces_at, ea_produces_at,
           ei_served_by, ea_served_by, ei_transmission, ea_transmission,
           ei_backed_by, ea_backed_by, proj_w_generator, proj_b_generator,
           proj_w_bus, proj_b_bus, proj_w_reserve, proj_b_reserve,
           hl_w_1, hl_b_1, ete_1, eae_w_1, att_w_1, lin_w_1, lin_b_1,
           hl_w_2, hl_b_2, ete_2, eae_w_2, att_w_2, lin_w_2, lin_b_2,
           hl_w_3, hl_b_3, ete_3, eae_w_3, att_w_3, lin_w_3, lin_b_3):
    xg = _mm_bias(x_generator, proj_w_generator, proj_b_generator, _NODE_TILE)
    xb = _mm_bias(x_bus, proj_w_bus, proj_b_bus, _NODE_TILE)
    xr = _mm_bias(x_reserve, proj_w_reserve, proj_b_reserve, _NODE_TILE)
    x = jnp.concatenate([xg, xb, xr], axis=0)

    edge_index = jnp.concatenate(
        [ei_produces_at, ei_served_by, ei_transmission, ei_backed_by], axis=1)
    src = edge_index[0]
    dst = edge_index[1]
    edge_attr = jnp.concatenate(
        [ea_produces_at, ea_served_by, ea_transmission, ea_backed_by], axis=0)

    h1 = jax.nn.relu(_heat_conv(x, src, dst, ete_1, edge_attr,
                                hl_w_1, hl_b_1, eae_w_1, att_w_1,
                                lin_w_1, lin_b_1))
    h2 = jax.nn.relu(_heat_conv(h1, src, dst, ete_2, edge_attr,
                                hl_w_2, hl_b_2, eae_w_2, att_w_2,
                                lin_w_2, lin_b_2)) + h1
    h3 = jax.nn.relu(_heat_conv(h2, src, dst, ete_3, edge_attr,
                                hl_w_3, hl_b_3, eae_w_3, att_w_3,
                                lin_w_3, lin_b_3)) + h2

    node_type = jnp.concatenate([
        jnp.zeros((_N_GEN,), jnp.int32),
        jnp.ones((_N_BUS,), jnp.int32),
        jnp.full((_N_RES,), 2, jnp.int32),
    ])
    return (h3, node_type)
